# baseline (speedup 1.0000x reference)
import functools

import jax
import jax.numpy as jnp
from jax import lax
from jax.experimental import pallas as pl
from jax.experimental.pallas import tpu as pltpu
from jax.experimental.pallas import tpu_sc as plsc

N_NODES = 10000
N_EDGES = 160000
D = 256
DH = 128

NPAD = 10240
ZROWS = NPAD // 16
CHUNK = 64
NCHUNK = 160
EPC = NCHUNK * CHUNK
EPAD = EPC * 16
RB = 4
IR = 8
UNROLL = 8
NOUTER = NCHUNK // UNROLL

BR = 400


def _sc_agg_body(srcp, dstp, ff, z2, z1, onesh,
                 aggl_o, aggr_o, deg0_o, deg1_o,
                 sv0, sv1, sv2, sv3, sv4, sv5, sv6, sv7,
                 dv0, dv1, dv2, dv3, dv4, dv5, dv6, dv7,
                 rows0, rows1, rows2, rows3, ones_v,
                 agg_sh, deg_sh,
                 is0, is1, is2, is3, is4, is5, is6, is7,
                 gs0, gs1, gs2, gs3, ss0, ss1, ss2, ss3, dsem):
  cid = lax.axis_index("c")
  sid = lax.axis_index("s")
  r0 = sid * ZROWS
  srcv = (sv0, sv1, sv2, sv3, sv4, sv5, sv6, sv7)
  dstv = (dv0, dv1, dv2, dv3, dv4, dv5, dv6, dv7)
  rows = (rows0, rows1, rows2, rows3)
  isems = (is0, is1, is2, is3, is4, is5, is6, is7)
  gsems = (gs0, gs1, gs2, gs3)
  ssems = (ss0, ss1, ss2, ss3)

  pltpu.sync_copy(z2, agg_sh.at[pl.ds(r0, ZROWS)])
  pltpu.sync_copy(z1, deg_sh.at[pl.ds(r0, ZROWS)])
  pltpu.sync_copy(onesh, ones_v)

  def idx_start(i, b):
    pltpu.async_copy(srcp.at[sid, i], srcv[b], isems[b])
    pltpu.async_copy(dstp.at[sid, i], dstv[b], isems[b])

  def idx_wait(b):
    pltpu.make_async_copy(srcp.at[0, 0], srcv[b], isems[b]).wait()
    pltpu.make_async_copy(dstp.at[0, 0], dstv[b], isems[b]).wait()
    for j in range(CHUNK // 16):
      sl = pl.ds(j * 16, 16)
      srcv[b][sl] = srcv[b][sl] * 2 + cid

  def gather_start(i, b, ib):
    pltpu.async_copy(ff.at[srcv[ib]], rows[b], gsems[b])

  def gather_wait(b, ib):
    pltpu.make_async_copy(ff.at[srcv[ib]], rows[b], gsems[b]).wait()

  def scatter_start(b, ib):
    pltpu.async_copy(rows[b], agg_sh.at[dstv[ib]], ssems[b], add=True)

  def scatter_wait(b, ib):
    pltpu.make_async_copy(rows[b], agg_sh.at[dstv[ib]], ssems[b]).wait()

  def deg_start(core, ib):
    @pl.when(cid == core)
    def _():
      pltpu.async_copy(ones_v, deg_sh.at[dstv[ib]], dsem, add=True)

  def deg_wait(core, ib):
    @pl.when(cid == core)
    def _():
      pltpu.make_async_copy(ones_v, deg_sh.at[dstv[ib]], dsem).wait()

  idx_start(0, 0)
  idx_start(1, 1)
  plsc.subcore_barrier()

  def outer(g, carry):
    for u in range(UNROLL):
      i = g * UNROLL + u
      b3 = u % RB
      pb3 = (u + RB - 1) % RB
      pb6 = (u + IR - 1) % IR

      def guard(cond_first, fn):
        if cond_first:
          fn()
        else:
          pl.when(g > 0)(fn)

      idx_wait(u)
      guard(u >= RB, lambda: scatter_wait(b3, (u + IR - RB) % IR))
      gather_start(i, b3, u)
      guard(u >= 1, lambda: gather_wait(pb3, pb6))
      guard(u >= 1, lambda: scatter_start(pb3, pb6))
      dcore = (u + 1) % 2
      guard(u >= 3, lambda: deg_wait(dcore, (u + IR - 3) % IR))
      guard(u >= 1, lambda: deg_start(dcore, pb6))
      if u < UNROLL - 2:
        idx_start(i + 2, (u + 2) % IR)
      else:
        @pl.when(g < NOUTER - 1)
        def _():
          idx_start(i + 2, (u + 2) % IR)
    return carry

  lax.fori_loop(0, NOUTER, outer, 0)

  L = NCHUNK - 1
  gather_wait(L % RB, L % IR)
  scatter_wait((L - 3) % RB, (L - 3) % IR)
  scatter_start(L % RB, L % IR)
  deg_wait((L - 1) % 2, (L - 1) % IR)
  deg_wait((L - 2) % 2, (L - 2) % IR)
  deg_start(L % 2, L % IR)
  deg_wait(L % 2, L % IR)
  scatter_wait((L - 2) % RB, (L - 2) % IR)
  scatter_wait((L - 1) % RB, (L - 1) % IR)
  scatter_wait(L % RB, L % IR)

  plsc.subcore_barrier()

  @pl.when(cid == 0)
  def _():
    pltpu.sync_copy(agg_sh.at[pl.ds(r0, ZROWS)], aggl_o.at[pl.ds(r0, ZROWS)])
    pltpu.sync_copy(deg_sh.at[pl.ds(r0, ZROWS)], deg0_o.at[pl.ds(r0, ZROWS)])

  @pl.when(cid == 1)
  def _():
    pltpu.sync_copy(agg_sh.at[pl.ds(r0, ZROWS)], aggr_o.at[pl.ds(r0, ZROWS)])
    pltpu.sync_copy(deg_sh.at[pl.ds(r0, ZROWS)], deg1_o.at[pl.ds(r0, ZROWS)])


@functools.lru_cache(maxsize=None)
def _build_sc_agg():
  return pl.kernel(
      _sc_agg_body,
      out_type=(
          jax.ShapeDtypeStruct((NPAD, DH), jnp.float32),
          jax.ShapeDtypeStruct((NPAD, DH), jnp.float32),
          jax.ShapeDtypeStruct((NPAD,), jnp.float32),
          jax.ShapeDtypeStruct((NPAD,), jnp.float32),
      ),
      mesh=plsc.VectorSubcoreMesh(core_axis_name="c", subcore_axis_name="s"),
      scratch_types=(
          pltpu.VMEM((CHUNK,), jnp.int32),
          pltpu.VMEM((CHUNK,), jnp.int32),
          pltpu.VMEM((CHUNK,), jnp.int32),
          pltpu.VMEM((CHUNK,), jnp.int32),
          pltpu.VMEM((CHUNK,), jnp.int32),
          pltpu.VMEM((CHUNK,), jnp.int32),
          pltpu.VMEM((CHUNK,), jnp.int32),
          pltpu.VMEM((CHUNK,), jnp.int32),
          pltpu.VMEM((CHUNK,), jnp.int32),
          pltpu.VMEM((CHUNK,), jnp.int32),
          pltpu.VMEM((CHUNK,), jnp.int32),
          pltpu.VMEM((CHUNK,), jnp.int32),
          pltpu.VMEM((CHUNK,), jnp.int32),
          pltpu.VMEM((CHUNK,), jnp.int32),
          pltpu.VMEM((CHUNK,), jnp.int32),
          pltpu.VMEM((CHUNK,), jnp.int32),
          pltpu.VMEM((CHUNK, DH), jnp.float32),
          pltpu.VMEM((CHUNK, DH), jnp.float32),
          pltpu.VMEM((CHUNK, DH), jnp.float32),
          pltpu.VMEM((CHUNK, DH), jnp.float32),
          pltpu.VMEM((CHUNK,), jnp.float32),
          pltpu.VMEM_SHARED((NPAD, DH), jnp.float32),
          pltpu.VMEM_SHARED((NPAD,), jnp.float32),
          pltpu.SemaphoreType.DMA,
          pltpu.SemaphoreType.DMA,
          pltpu.SemaphoreType.DMA,
          pltpu.SemaphoreType.DMA,
          pltpu.SemaphoreType.DMA,
          pltpu.SemaphoreType.DMA,
          pltpu.SemaphoreType.DMA,
          pltpu.SemaphoreType.DMA,
          pltpu.SemaphoreType.DMA,
          pltpu.SemaphoreType.DMA,
          pltpu.SemaphoreType.DMA,
          pltpu.SemaphoreType.DMA,
          pltpu.SemaphoreType.DMA,
          pltpu.SemaphoreType.DMA,
          pltpu.SemaphoreType.DMA,
          pltpu.SemaphoreType.DMA,
          pltpu.SemaphoreType.DMA,
      ),
  )


def _tc_base_body(feat_ref, wr_ref, br_ref, ug_ref, bg_ref, un_ref, bn_ref,
                  base_ref, p1_ref):
  f = feat_ref[...]
  dot = functools.partial(jnp.dot, preferred_element_type=jnp.float32,
                          precision=lax.Precision.DEFAULT)
  lg = dot(f, wr_ref[...]) + br_ref[...]
  dlt = lg[:, 1:2] - lg[:, 0:1]
  p1 = 1.0 / (1.0 + jnp.exp(-dlt))
  p0 = 1.0 - p1
  base_ref[...] = (p0 * (dot(f, ug_ref[...]) + bg_ref[...])
                   + p1 * (dot(f, un_ref[...]) + bn_ref[...])).astype(jnp.bfloat16)
  p1_ref[...] = p1


_tc_base = pl.pallas_call(
    _tc_base_body,
    grid=(N_NODES // BR,),
    in_specs=[
        pl.BlockSpec((BR, D), lambda i: (i, 0)),
        pl.BlockSpec((D, 2), lambda i: (0, 0)),
        pl.BlockSpec((1, 2), lambda i: (0, 0)),
        pl.BlockSpec((D, D), lambda i: (0, 0)),
        pl.BlockSpec((1, D), lambda i: (0, 0)),
        pl.BlockSpec((D, D), lambda i: (0, 0)),
        pl.BlockSpec((1, D), lambda i: (0, 0)),
    ],
    out_specs=[
        pl.BlockSpec((BR, D), lambda i: (i, 0)),
        pl.BlockSpec((BR, 1), lambda i: (i, 0)),
    ],
    out_shape=[
        jax.ShapeDtypeStruct((N_NODES, D), jnp.bfloat16),
        jax.ShapeDtypeStruct((N_NODES, 1), jnp.float32),
    ],
)


def _tc_comb_body(aggl_ref, aggr_ref, deg0_ref, deg1_ref, base_ref, p1_ref,
                  wg_ref, wn_ref, out_ref):
  rdeg = 1.0 / jnp.maximum(deg0_ref[...] + deg1_ref[...], 1.0)
  al = aggl_ref[...] * rdeg
  ar = aggr_ref[...] * rdeg
  p1 = p1_ref[...]
  dot = functools.partial(jnp.dot, preferred_element_type=jnp.float32,
                          precision=lax.Precision.DEFAULT)
  gcn = dot(al, wg_ref[0:DH, :]) + dot(ar, wg_ref[DH:D, :])
  ncn = jnp.maximum(dot(al, wn_ref[0:DH, :]) + dot(ar, wn_ref[DH:D, :]), 0.0)
  out_ref[...] = (base_ref[...].astype(jnp.float32)
                  + (1.0 - p1) * gcn + p1 * ncn)


_tc_comb = pl.pallas_call(
    _tc_comb_body,
    grid=(N_NODES // BR,),
    in_specs=[
        pl.BlockSpec((BR, DH), lambda i: (i, 0)),
        pl.BlockSpec((BR, DH), lambda i: (i, 0)),
        pl.BlockSpec((BR, 1), lambda i: (i, 0)),
        pl.BlockSpec((BR, 1), lambda i: (i, 0)),
        pl.BlockSpec((BR, D), lambda i: (i, 0)),
        pl.BlockSpec((BR, 1), lambda i: (i, 0)),
        pl.BlockSpec((D, D), lambda i: (0, 0)),
        pl.BlockSpec((D, D), lambda i: (0, 0)),
    ],
    out_specs=pl.BlockSpec((BR, D), lambda i: (i, 0)),
    out_shape=jax.ShapeDtypeStruct((N_NODES, D), jnp.float32),
)


def kernel(feat, edge_index, W_r, b_r, W_gcn, U_gcn, b_gcn, W_ncn, U_ncn, b_ncn):
  src = edge_index[0]
  dst = edge_index[1]

  pad = EPAD - N_EDGES
  srcp = jnp.concatenate(
      [src, jnp.zeros((pad,), jnp.int32)]).reshape(16, NCHUNK, CHUNK)
  dstp = jnp.concatenate(
      [dst, jnp.full((pad,), N_NODES, jnp.int32)]).reshape(16, NCHUNK, CHUNK)

  ff = feat.reshape(2 * N_NODES, DH)

  z2 = jnp.zeros((ZROWS, DH), jnp.float32)
  z1 = jnp.zeros((ZROWS,), jnp.float32)
  onesh = jnp.ones((CHUNK,), jnp.float32)

  aggl, aggr, deg0, deg1 = _build_sc_agg()(srcp, dstp, ff, z2, z1, onesh)

  base, p1 = _tc_base(feat, W_r, b_r.reshape(1, 2), U_gcn, b_gcn.reshape(1, D),
                      U_ncn, b_ncn.reshape(1, D))

  out = _tc_comb(aggl, aggr, deg0.reshape(NPAD, 1), deg1.reshape(NPAD, 1),
                 base, p1, W_gcn, W_ncn)
  return out


# speedup vs baseline: 1.6351x; 1.6351x over previous
import functools

import jax
import jax.numpy as jnp
from jax import lax
from jax.experimental import pallas as pl
from jax.experimental.pallas import tpu as pltpu
from jax.experimental.pallas import tpu_sc as plsc

N_NODES = 10000
N_EDGES = 160000
D = 256
DH = 128

NPAD = 10240
ZROWS = NPAD // 16
CHUNK = 120
NCHUNK = 84
EPC = NCHUNK * CHUNK
EPAD = EPC * 16
RB = 3
IR = 6
UNROLL = 6
NOUTER = NCHUNK // UNROLL

BR = 2000


def _sc_agg_body(srcp, dstp, ff, z2, z1, onesh,
                 aggl_o, aggr_o, deg0_o, deg1_o,
                 sv0, sv1, sv2, sv3, sv4, sv5,
                 dv0, dv1, dv2, dv3, dv4, dv5,
                 rows0, rows1, rows2, ones_v,
                 agg_sh, deg_sh,
                 is0, is1, is2, is3, is4, is5,
                 gs0, gs1, gs2, ss0, ss1, ss2, dsem):
  cid = lax.axis_index("c")
  sid = lax.axis_index("s")
  r0 = sid * ZROWS
  srcv = (sv0, sv1, sv2, sv3, sv4, sv5)
  dstv = (dv0, dv1, dv2, dv3, dv4, dv5)
  rows = (rows0, rows1, rows2)
  isems = (is0, is1, is2, is3, is4, is5)
  gsems = (gs0, gs1, gs2)
  ssems = (ss0, ss1, ss2)

  pltpu.sync_copy(z2, agg_sh.at[pl.ds(r0, ZROWS)])
  pltpu.sync_copy(z1, deg_sh.at[pl.ds(r0, ZROWS)])
  pltpu.sync_copy(onesh, ones_v)

  def idx_start(i, b):
    pltpu.async_copy(srcp.at[sid, i], srcv[b], isems[b])
    pltpu.async_copy(dstp.at[sid, i], dstv[b], isems[b])

  def idx_wait(b):
    pltpu.make_async_copy(srcp.at[0, 0], srcv[b], isems[b]).wait()
    pltpu.make_async_copy(dstp.at[0, 0], dstv[b], isems[b]).wait()
    for j in range(CHUNK // 16):
      sl = pl.ds(j * 16, 16)
      srcv[b][sl] = srcv[b][sl] * 2 + cid

  def gather_start(i, b, ib):
    pltpu.async_copy(ff.at[srcv[ib]], rows[b], gsems[b])

  def gather_wait(b, ib):
    pltpu.make_async_copy(ff.at[srcv[ib]], rows[b], gsems[b]).wait()

  def scatter_start(b, ib):
    pltpu.async_copy(rows[b], agg_sh.at[dstv[ib]], ssems[b], add=True)

  def scatter_wait(b, ib):
    pltpu.make_async_copy(rows[b], agg_sh.at[dstv[ib]], ssems[b]).wait()

  def deg_start(core, ib):
    @pl.when(cid == core)
    def _():
      pltpu.async_copy(ones_v, deg_sh.at[dstv[ib]], dsem, add=True)

  def deg_wait(core, ib):
    @pl.when(cid == core)
    def _():
      pltpu.make_async_copy(ones_v, deg_sh.at[dstv[ib]], dsem).wait()

  idx_start(0, 0)
  idx_start(1, 1)
  plsc.subcore_barrier()

  def outer(g, carry):
    for u in range(UNROLL):
      i = g * UNROLL + u
      b3 = u % RB
      pb3 = (u + RB - 1) % RB
      pb6 = (u + IR - 1) % IR

      def guard(cond_first, fn):
        if cond_first:
          fn()
        else:
          pl.when(g > 0)(fn)

      idx_wait(u)
      guard(u >= 3, lambda: scatter_wait(b3, (u + IR - 3) % IR))
      gather_start(i, b3, u)
      guard(u >= 1, lambda: gather_wait(pb3, pb6))
      guard(u >= 1, lambda: scatter_start(pb3, pb6))
      dcore = (u + 1) % 2
      guard(u >= 3, lambda: deg_wait(dcore, (u + IR - 3) % IR))
      guard(u >= 1, lambda: deg_start(dcore, pb6))
      if u < 4:
        idx_start(i + 2, (u + 2) % IR)
      else:
        @pl.when(g < NOUTER - 1)
        def _():
          idx_start(i + 2, (u + 2) % IR)
    return carry

  lax.fori_loop(0, NOUTER, outer, 0)

  L = NCHUNK - 1
  gather_wait(L % RB, L % IR)
  scatter_wait((L - 2) % RB, (L - 2) % IR)
  scatter_start(L % RB, L % IR)
  deg_wait(0, (L - 1) % IR)
  deg_wait(1, (L - 2) % IR)
  deg_start(1, L % IR)
  deg_wait(1, L % IR)
  scatter_wait((L - 1) % RB, (L - 1) % IR)
  scatter_wait(L % RB, L % IR)

  plsc.subcore_barrier()

  @pl.when(cid == 0)
  def _():
    pltpu.sync_copy(agg_sh.at[pl.ds(r0, ZROWS)], aggl_o.at[pl.ds(r0, ZROWS)])
    pltpu.sync_copy(deg_sh.at[pl.ds(r0, ZROWS)], deg0_o.at[pl.ds(r0, ZROWS)])

  @pl.when(cid == 1)
  def _():
    pltpu.sync_copy(agg_sh.at[pl.ds(r0, ZROWS)], aggr_o.at[pl.ds(r0, ZROWS)])
    pltpu.sync_copy(deg_sh.at[pl.ds(r0, ZROWS)], deg1_o.at[pl.ds(r0, ZROWS)])


@functools.lru_cache(maxsize=None)
def _build_sc_agg():
  return pl.kernel(
      _sc_agg_body,
      out_type=(
          jax.ShapeDtypeStruct((NPAD, DH), jnp.float32),
          jax.ShapeDtypeStruct((NPAD, DH), jnp.float32),
          jax.ShapeDtypeStruct((NPAD,), jnp.float32),
          jax.ShapeDtypeStruct((NPAD,), jnp.float32),
      ),
      mesh=plsc.VectorSubcoreMesh(core_axis_name="c", subcore_axis_name="s"),
      scratch_types=(
          pltpu.VMEM((CHUNK,), jnp.int32),
          pltpu.VMEM((CHUNK,), jnp.int32),
          pltpu.VMEM((CHUNK,), jnp.int32),
          pltpu.VMEM((CHUNK,), jnp.int32),
          pltpu.VMEM((CHUNK,), jnp.int32),
          pltpu.VMEM((CHUNK,), jnp.int32),
          pltpu.VMEM((CHUNK,), jnp.int32),
          pltpu.VMEM((CHUNK,), jnp.int32),
          pltpu.VMEM((CHUNK,), jnp.int32),
          pltpu.VMEM((CHUNK,), jnp.int32),
          pltpu.VMEM((CHUNK,), jnp.int32),
          pltpu.VMEM((CHUNK,), jnp.int32),
          pltpu.VMEM((CHUNK, DH), jnp.float32),
          pltpu.VMEM((CHUNK, DH), jnp.float32),
          pltpu.VMEM((CHUNK, DH), jnp.float32),
          pltpu.VMEM((CHUNK,), jnp.float32),
          pltpu.VMEM_SHARED((NPAD, DH), jnp.float32),
          pltpu.VMEM_SHARED((NPAD,), jnp.float32),
          pltpu.SemaphoreType.DMA,
          pltpu.SemaphoreType.DMA,
          pltpu.SemaphoreType.DMA,
          pltpu.SemaphoreType.DMA,
          pltpu.SemaphoreType.DMA,
          pltpu.SemaphoreType.DMA,
          pltpu.SemaphoreType.DMA,
          pltpu.SemaphoreType.DMA,
          pltpu.SemaphoreType.DMA,
          pltpu.SemaphoreType.DMA,
          pltpu.SemaphoreType.DMA,
          pltpu.SemaphoreType.DMA,
          pltpu.SemaphoreType.DMA,
      ),
  )


def _tc_base_body(feat_ref, wr_ref, br_ref, ug_ref, bg_ref, un_ref, bn_ref,
                  base_ref, p1_ref):
  f = feat_ref[...]
  dot = functools.partial(jnp.dot, preferred_element_type=jnp.float32,
                          precision=lax.Precision.DEFAULT)
  lg = dot(f, wr_ref[...]) + br_ref[...]
  dlt = lg[:, 1:2] - lg[:, 0:1]
  p1 = 1.0 / (1.0 + jnp.exp(-dlt))
  p0 = 1.0 - p1
  base_ref[...] = (p0 * (dot(f, ug_ref[...]) + bg_ref[...])
                   + p1 * (dot(f, un_ref[...]) + bn_ref[...])).astype(jnp.bfloat16)
  p1_ref[...] = p1


_tc_base = pl.pallas_call(
    _tc_base_body,
    grid=(N_NODES // BR,),
    in_specs=[
        pl.BlockSpec((BR, D), lambda i: (i, 0)),
        pl.BlockSpec((D, 2), lambda i: (0, 0)),
        pl.BlockSpec((1, 2), lambda i: (0, 0)),
        pl.BlockSpec((D, D), lambda i: (0, 0)),
        pl.BlockSpec((1, D), lambda i: (0, 0)),
        pl.BlockSpec((D, D), lambda i: (0, 0)),
        pl.BlockSpec((1, D), lambda i: (0, 0)),
    ],
    out_specs=[
        pl.BlockSpec((BR, D), lambda i: (i, 0)),
        pl.BlockSpec((BR, 1), lambda i: (i, 0)),
    ],
    out_shape=[
        jax.ShapeDtypeStruct((N_NODES, D), jnp.bfloat16),
        jax.ShapeDtypeStruct((N_NODES, 1), jnp.float32),
    ],
)


def _tc_comb_body(aggl_ref, aggr_ref, deg0_ref, deg1_ref, base_ref, p1_ref,
                  wg_ref, wn_ref, out_ref):
  rdeg = 1.0 / jnp.maximum(deg0_ref[...] + deg1_ref[...], 1.0)
  al = aggl_ref[...] * rdeg
  ar = aggr_ref[...] * rdeg
  p1 = p1_ref[...]
  dot = functools.partial(jnp.dot, preferred_element_type=jnp.float32,
                          precision=lax.Precision.DEFAULT)
  gcn = dot(al, wg_ref[0:DH, :]) + dot(ar, wg_ref[DH:D, :])
  ncn = jnp.maximum(dot(al, wn_ref[0:DH, :]) + dot(ar, wn_ref[DH:D, :]), 0.0)
  out_ref[...] = (base_ref[...].astype(jnp.float32)
                  + (1.0 - p1) * gcn + p1 * ncn)


_tc_comb = pl.pallas_call(
    _tc_comb_body,
    grid=(N_NODES // BR,),
    in_specs=[
        pl.BlockSpec((BR, DH), lambda i: (i, 0)),
        pl.BlockSpec((BR, DH), lambda i: (i, 0)),
        pl.BlockSpec((BR, 1), lambda i: (i, 0)),
        pl.BlockSpec((BR, 1), lambda i: (i, 0)),
        pl.BlockSpec((BR, D), lambda i: (i, 0)),
        pl.BlockSpec((BR, 1), lambda i: (i, 0)),
        pl.BlockSpec((D, D), lambda i: (0, 0)),
        pl.BlockSpec((D, D), lambda i: (0, 0)),
    ],
    out_specs=pl.BlockSpec((BR, D), lambda i: (i, 0)),
    out_shape=jax.ShapeDtypeStruct((N_NODES, D), jnp.float32),
)


def kernel(feat, edge_index, W_r, b_r, W_gcn, U_gcn, b_gcn, W_ncn, U_ncn, b_ncn):
  src = edge_index[0]
  dst = edge_index[1]

  pad = EPAD - N_EDGES
  srcp = jnp.concatenate(
      [src, jnp.zeros((pad,), jnp.int32)]).reshape(16, NCHUNK, CHUNK)
  dstp = jnp.concatenate(
      [dst, jnp.full((pad,), N_NODES, jnp.int32)]).reshape(16, NCHUNK, CHUNK)

  ff = feat.reshape(2 * N_NODES, DH)

  z2 = jnp.zeros((ZROWS, DH), jnp.float32)
  z1 = jnp.zeros((ZROWS,), jnp.float32)
  onesh = jnp.ones((CHUNK,), jnp.float32)

  aggl, aggr, deg0, deg1 = _build_sc_agg()(srcp, dstp, ff, z2, z1, onesh)

  base, p1 = _tc_base(feat, W_r, b_r.reshape(1, 2), U_gcn, b_gcn.reshape(1, D),
                      U_ncn, b_ncn.reshape(1, D))

  out = _tc_comb(aggl, aggr, deg0.reshape(NPAD, 1), deg1.reshape(NPAD, 1),
                 base, p1, W_gcn, W_ncn)
  return out
